# baseline (speedup 1.0000x reference)
import functools
import jax
import jax.numpy as jnp
from jax import lax
from jax.experimental import pallas as pl
from jax.experimental.pallas import tpu as pltpu
from jax.experimental.pallas import tpu_sc as plsc

N = 10000
E = 320000
D = 128
T = 4
L = 16
NW = 32
EPW = E // NW
CH = 128
NFULL = EPW // CH
TAIL = EPW - NFULL * CH
NPAIR = NFULL // 2
DV = D // L


def _make_kernel():
  mesh = plsc.VectorSubcoreMesh(core_axis_name="c", subcore_axis_name="s")

  @functools.partial(
      pl.kernel,
      mesh=mesh,
      out_type=jax.ShapeDtypeStruct((E * T,), jnp.float32),
      compiler_params=pltpu.CompilerParams(needs_layout_passes=False),
      scratch_types=[
          pltpu.VMEM((EPW,), jnp.int32),
          pltpu.VMEM((EPW,), jnp.int32),
          pltpu.VMEM((CH, D), jnp.float32),
          pltpu.VMEM((CH, D), jnp.float32),
          pltpu.VMEM((CH, D), jnp.float32),
          pltpu.VMEM((CH, D), jnp.float32),
          pltpu.VMEM((CH * T,), jnp.float32),
          pltpu.VMEM((CH * T,), jnp.float32),
          pltpu.VMEM((T, DV, L), jnp.float32),
          pltpu.VMEM((L,), jnp.float32),
          pltpu.SemaphoreType.DMA,
          pltpu.SemaphoreType.DMA,
          pltpu.SemaphoreType.DMA,
          pltpu.SemaphoreType.DMA,
          pltpu.SemaphoreType.DMA,
          pltpu.SemaphoreType.DMA,
      ],
  )
  def k(table_hbm, src_hbm, dst_hbm, wr_hbm, binit_hbm, out_hbm,
        sidx, didx, srA, drA, srB, drB, outA, outB, wr_v, b_v,
        gsA, gdA, gsB, gdB, oA, oB):
    wid = lax.axis_index("s") * 2 + lax.axis_index("c")
    base = wid * EPW

    pltpu.sync_copy(wr_hbm, wr_v)
    pltpu.sync_copy(binit_hbm, b_v)
    pltpu.sync_copy(src_hbm.at[pl.ds(base, EPW)], sidx)
    pltpu.sync_copy(dst_hbm.at[pl.ds(base, EPW)], didx)

    wvec = [[wr_v[t, i, :] for i in range(DV)] for t in range(T)]
    btile = b_v[:]

    def issue(k_chunk, sr, dr, gs, gd):
      off = k_chunk * CH
      pltpu.async_copy(table_hbm.at[sidx.at[pl.ds(off, CH)]], sr, gs)
      pltpu.async_copy(table_hbm.at[didx.at[pl.ds(off, CH)]], dr, gd)

    def wait_gathers(sr, dr, gs, gd):
      pltpu.make_async_copy(table_hbm.at[sidx.at[pl.ds(0, CH)]], sr, gs).wait()
      pltpu.make_async_copy(table_hbm.at[didx.at[pl.ds(0, CH)]], dr, gd).wait()

    def compute(sr, dr, ob, ngrp):
      def init_body(g, _):
        ob[pl.ds(g * L, L)] = btile
        return _

      lax.fori_loop(0, ngrp, init_body, None, unroll=4)

      def edge_body(i, _):
        for j in range(T):
          e = T * i + j
          h = [
              jnp.maximum(
                  sr[e, L * v:L * (v + 1)] + dr[e, L * v:L * (v + 1)], 0.0)
              for v in range(DV)
          ]
          for t in range(T):
            acc = h[0] * wvec[t][0]
            for v in range(1, DV):
              acc = acc + h[v] * wvec[t][v]
            idxv = jnp.full((L,), e * T + t, jnp.int32)
            plsc.addupdate_scatter(ob, [idxv], acc)
        return _

      lax.fori_loop(0, ngrp, edge_body, None, unroll=1)

    def out_start(k_chunk, ob, sem):
      pltpu.async_copy(
          ob, out_hbm.at[pl.ds((base + k_chunk * CH) * T, CH * T)], sem)

    def out_wait(ob, sem):
      pltpu.make_async_copy(
          ob, out_hbm.at[pl.ds(base * T, CH * T)], sem).wait()

    issue(0, srA, drA, gsA, gdA)
    issue(1, srB, drB, gsB, gdB)

    def pair_body(i, _):
      k0 = 2 * i
      wait_gathers(srA, drA, gsA, gdA)

      @pl.when(i > 0)
      def _wA():
        out_wait(outA, oA)

      compute(srA, drA, outA, CH // T)
      out_start(k0, outA, oA)

      @pl.when(i < NPAIR - 1)
      def _iA():
        issue(k0 + 2, srA, drA, gsA, gdA)

      wait_gathers(srB, drB, gsB, gdB)

      @pl.when(i > 0)
      def _wB():
        out_wait(outB, oB)

      compute(srB, drB, outB, CH // T)
      out_start(k0 + 1, outB, oB)

      @pl.when(i < NPAIR - 1)
      def _iB():
        issue(k0 + 3, srB, drB, gsB, gdB)

      return _

    lax.fori_loop(0, NPAIR, pair_body, None)

    toff = NFULL * CH
    pltpu.async_copy(
        table_hbm.at[sidx.at[pl.ds(toff, TAIL)]], srA.at[pl.ds(0, TAIL)], gsA)
    pltpu.async_copy(
        table_hbm.at[didx.at[pl.ds(toff, TAIL)]], drA.at[pl.ds(0, TAIL)], gdA)
    pltpu.make_async_copy(
        table_hbm.at[sidx.at[pl.ds(toff, TAIL)]], srA.at[pl.ds(0, TAIL)],
        gsA).wait()
    pltpu.make_async_copy(
        table_hbm.at[didx.at[pl.ds(toff, TAIL)]], drA.at[pl.ds(0, TAIL)],
        gdA).wait()
    out_wait(outA, oA)
    compute(srA, drA, outA, TAIL // T)
    out_wait(outB, oB)
    pltpu.sync_copy(
        outA.at[pl.ds(0, TAIL * T)],
        out_hbm.at[pl.ds((base + toff) * T, TAIL * T)])

  return k


_kern = _make_kernel()


def kernel(encoded_graph, edge_index, W, b):
  ei = edge_index.astype(jnp.int32)
  src = ei[0]
  dst = ei[1]
  wr = W.T.reshape(T, DV, L)
  binit = jnp.tile(b, L // T)
  return _kern(encoded_graph, src, dst, wr, binit).reshape(E, T)


# speedup vs baseline: 1.9682x; 1.9682x over previous
import functools
import jax
import jax.numpy as jnp
from jax import lax
from jax.experimental import pallas as pl
from jax.experimental.pallas import tpu as pltpu
from jax.experimental.pallas import tpu_sc as plsc

N = 10000
E = 320000
D = 128
T = 4
L = 16
NW = 32
EPW = E // NW
CH = 128
NFULL = EPW // CH
TAIL = EPW - NFULL * CH
NPAIR = NFULL // 2
DV = D // L


def _make_kernel():
  mesh = plsc.VectorSubcoreMesh(core_axis_name="c", subcore_axis_name="s")

  @functools.partial(
      pl.kernel,
      mesh=mesh,
      out_type=jax.ShapeDtypeStruct((E * T,), jnp.float32),
      compiler_params=pltpu.CompilerParams(needs_layout_passes=False),
      scratch_types=[
          pltpu.VMEM((EPW,), jnp.int32),
          pltpu.VMEM((EPW,), jnp.int32),
          pltpu.VMEM((CH, D), jnp.float32),
          pltpu.VMEM((CH, D), jnp.float32),
          pltpu.VMEM((CH, D), jnp.float32),
          pltpu.VMEM((CH, D), jnp.float32),
          pltpu.VMEM((CH * T,), jnp.float32),
          pltpu.VMEM((CH * T,), jnp.float32),
          pltpu.VMEM((T, DV, L), jnp.float32),
          pltpu.VMEM((L,), jnp.float32),
          pltpu.VMEM((L * L,), jnp.float32),
          pltpu.VMEM((L * L,), jnp.float32),
          pltpu.SemaphoreType.DMA,
          pltpu.SemaphoreType.DMA,
          pltpu.SemaphoreType.DMA,
          pltpu.SemaphoreType.DMA,
          pltpu.SemaphoreType.DMA,
          pltpu.SemaphoreType.DMA,
      ],
  )
  def k(table_hbm, src_hbm, dst_hbm, wr_hbm, binit_hbm, out_hbm,
        sidx, didx, srA, drA, srB, drB, outA, outB, wr_v, b_v, pb0, pb1,
        gsA, gdA, gsB, gdB, oA, oB):
    wid = lax.axis_index("s") * 2 + lax.axis_index("c")
    base = wid * EPW

    pltpu.sync_copy(wr_hbm, wr_v)
    pltpu.sync_copy(binit_hbm, b_v)
    pltpu.sync_copy(src_hbm.at[pl.ds(base, EPW)], sidx)
    pltpu.sync_copy(dst_hbm.at[pl.ds(base, EPW)], didx)

    wvec = [[wr_v[t, i, :] for i in range(DV)] for t in range(T)]
    btile = b_v[:]
    lane16 = lax.iota(jnp.int32, L) * L

    def issue(k_chunk, sr, dr, gs, gd):
      off = k_chunk * CH
      pltpu.async_copy(table_hbm.at[sidx.at[pl.ds(off, CH)]], sr, gs)
      pltpu.async_copy(table_hbm.at[didx.at[pl.ds(off, CH)]], dr, gd)

    def wait_gathers(sr, dr, gs, gd):
      pltpu.make_async_copy(table_hbm.at[sidx.at[pl.ds(0, CH)]], sr, gs).wait()
      pltpu.make_async_copy(table_hbm.at[didx.at[pl.ds(0, CH)]], dr, gd).wait()

    def compute(sr, dr, ob, ngrp):
      def group(g, pb):
        for j in range(T):
          e = T * g + j
          h = [
              jnp.maximum(
                  sr[e, L * v:L * (v + 1)] + dr[e, L * v:L * (v + 1)], 0.0)
              for v in range(DV)
          ]
          for t in range(T):
            acc = h[0] * wvec[t][0]
            for v in range(1, DV):
              acc = acc + h[v] * wvec[t][v]
            plsc.store_scatter(pb, [lane16 + (j * T + t)], acc)
        ov = pb[0:L] + btile
        for r in range(1, L):
          ov = ov + pb[L * r:L * (r + 1)]
        ob[pl.ds(g * L, L)] = ov

      def pair(i, _):
        group(2 * i, pb0)
        group(2 * i + 1, pb1)
        return _

      lax.fori_loop(0, ngrp // 2, pair, None, unroll=1)

    def out_start(k_chunk, ob, sem):
      pltpu.async_copy(
          ob, out_hbm.at[pl.ds((base + k_chunk * CH) * T, CH * T)], sem)

    def out_wait(ob, sem):
      pltpu.make_async_copy(
          ob, out_hbm.at[pl.ds(base * T, CH * T)], sem).wait()

    issue(0, srA, drA, gsA, gdA)
    issue(1, srB, drB, gsB, gdB)

    def pair_body(i, _):
      k0 = 2 * i
      wait_gathers(srA, drA, gsA, gdA)

      @pl.when(i > 0)
      def _wA():
        out_wait(outA, oA)

      compute(srA, drA, outA, CH // T)
      out_start(k0, outA, oA)

      @pl.when(i < NPAIR - 1)
      def _iA():
        issue(k0 + 2, srA, drA, gsA, gdA)

      wait_gathers(srB, drB, gsB, gdB)

      @pl.when(i > 0)
      def _wB():
        out_wait(outB, oB)

      compute(srB, drB, outB, CH // T)
      out_start(k0 + 1, outB, oB)

      @pl.when(i < NPAIR - 1)
      def _iB():
        issue(k0 + 3, srB, drB, gsB, gdB)

      return _

    lax.fori_loop(0, NPAIR, pair_body, None)

    toff = NFULL * CH
    pltpu.async_copy(
        table_hbm.at[sidx.at[pl.ds(toff, TAIL)]], srA.at[pl.ds(0, TAIL)], gsA)
    pltpu.async_copy(
        table_hbm.at[didx.at[pl.ds(toff, TAIL)]], drA.at[pl.ds(0, TAIL)], gdA)
    pltpu.make_async_copy(
        table_hbm.at[sidx.at[pl.ds(toff, TAIL)]], srA.at[pl.ds(0, TAIL)],
        gsA).wait()
    pltpu.make_async_copy(
        table_hbm.at[didx.at[pl.ds(toff, TAIL)]], drA.at[pl.ds(0, TAIL)],
        gdA).wait()
    out_wait(outA, oA)
    compute(srA, drA, outA, TAIL // T)
    out_wait(outB, oB)
    pltpu.sync_copy(
        outA.at[pl.ds(0, TAIL * T)],
        out_hbm.at[pl.ds((base + toff) * T, TAIL * T)])

  return k


_kern = _make_kernel()


def kernel(encoded_graph, edge_index, W, b):
  ei = edge_index.astype(jnp.int32)
  src = ei[0]
  dst = ei[1]
  wr = W.T.reshape(T, DV, L)
  binit = jnp.tile(b, L // T)
  return _kern(encoded_graph, src, dst, wr, binit).reshape(E, T)
